# baseline (speedup 1.0000x reference)
import jax
import jax.numpy as jnp
from jax.experimental import pallas as pl

R, C = 128, 32768
BR = 8


def _body(x_ref, m_ref, o_ref):
    o_ref[...] = x_ref[...] * m_ref[...]


def kernel(x, mask):
    return pl.pallas_call(
        _body,
        out_shape=jax.ShapeDtypeStruct((R, C), x.dtype),
        grid=(R // BR,),
        in_specs=[
            pl.BlockSpec((BR, C), lambda j: (j, 0)),
            pl.BlockSpec((BR, 1), lambda j: (j, 0)),
        ],
        out_specs=pl.BlockSpec((BR, C), lambda j: (j, 0)),
    )(x, mask[:, None])


# speedup vs baseline: 1.5927x; 1.5927x over previous
import jax
import jax.numpy as jnp
from jax.experimental import pallas as pl

R, C = 128, 32768
BC = 16384


def _body(x_ref, m_ref, o_ref):
    o_ref[...] = x_ref[...] * m_ref[...]


def kernel(x, mask):
    return pl.pallas_call(
        _body,
        out_shape=jax.ShapeDtypeStruct((R, C), x.dtype),
        grid=(C // BC,),
        in_specs=[
            pl.BlockSpec((R, BC), lambda j: (0, j)),
            pl.BlockSpec((R, 1), lambda j: (0, 0)),
        ],
        out_specs=pl.BlockSpec((R, BC), lambda j: (0, j)),
    )(x, mask[:, None])


# speedup vs baseline: 1.6115x; 1.0118x over previous
import jax
import jax.numpy as jnp
from jax.experimental import pallas as pl

R, C = 128, 32768
BR = 64


def _body(x_ref, m_ref, o_ref):
    o_ref[...] = x_ref[...] * m_ref[...]


def kernel(x, mask):
    return pl.pallas_call(
        _body,
        out_shape=jax.ShapeDtypeStruct((R, C), x.dtype),
        grid=(R // BR,),
        in_specs=[
            pl.BlockSpec((BR, C), lambda j: (j, 0)),
            pl.BlockSpec((BR, 1), lambda j: (j, 0)),
        ],
        out_specs=pl.BlockSpec((BR, C), lambda j: (j, 0)),
    )(x, mask[:, None])
